# baseline (speedup 1.0000x reference)
import functools

import jax
import jax.numpy as jnp
from jax import lax
from jax.experimental import pallas as pl
from jax.experimental.pallas import tpu as pltpu
from jax.experimental.pallas import tpu_sc as plsc

_NUM_CORES = 2
_NUM_SUBCORES = 16
_NUM_WORKERS = _NUM_CORES * _NUM_SUBCORES


def _copy_body(table_hbm, out_hbm, sem):
    wid = lax.axis_index("s") * _NUM_CORES + lax.axis_index("c")
    rows = table_hbm.shape[0]
    chunk = (rows // _NUM_WORKERS) // 8 * 8
    rem = rows - chunk * _NUM_WORKERS
    base = wid * chunk
    main = pltpu.async_copy(
        table_hbm.at[pl.ds(base, chunk)],
        out_hbm.at[pl.ds(base, chunk)],
        sem,
    )
    if rem:
        tail_base = chunk * _NUM_WORKERS

        @pl.when(wid == 0)
        def _():
            pltpu.async_copy(
                table_hbm.at[pl.ds(tail_base, rem)],
                out_hbm.at[pl.ds(tail_base, rem)],
                sem,
            ).wait()

    main.wait()


def kernel(embed_node):
    mesh = plsc.VectorSubcoreMesh(
        core_axis_name="c", subcore_axis_name="s", num_cores=_NUM_CORES
    )
    fn = pl.kernel(
        _copy_body,
        out_type=jax.ShapeDtypeStruct(embed_node.shape, embed_node.dtype),
        mesh=mesh,
        scratch_types=[pltpu.SemaphoreType.DMA],
    )
    return fn(embed_node)


# speedup vs baseline: 17.0003x; 17.0003x over previous
import jax
import jax.numpy as jnp
from jax import lax
from jax.experimental import pallas as pl
from jax.experimental.pallas import tpu as pltpu
from jax.experimental.pallas import tpu_sc as plsc

_NUM_CORES = 2
_NUM_SUBCORES = 16
_NUM_WORKERS = _NUM_CORES * _NUM_SUBCORES

_WIDTH = 128
_NBUF = 2
_BLK = 488
_NBLK = 8
_CHUNK = _BLK * _NBLK
_MAIN = _CHUNK * _NUM_WORKERS
_TAIL = 8


def _copy_body(table_hbm, out_hbm, bufs, rsems, wsems):
    wid = lax.axis_index("s") * _NUM_CORES + lax.axis_index("c")
    base = wid * _CHUNK

    def read(i):
        return pltpu.async_copy(
            table_hbm.at[pl.ds(base + i * _BLK, _BLK)],
            bufs[i % _NBUF],
            rsems[i % _NBUF],
        )

    def write(i):
        return pltpu.async_copy(
            bufs[i % _NBUF],
            out_hbm.at[pl.ds(base + i * _BLK, _BLK)],
            wsems[i % _NBUF],
        )

    rd = [None] * _NBLK
    wr = [None] * _NBLK
    for i in range(_NBUF):
        rd[i] = read(i)
    for i in range(_NBLK):
        rd[i].wait()
        wr[i] = write(i)
        if i + _NBUF < _NBLK:
            wr[i].wait()
            rd[i + _NBUF] = read(i + _NBUF)
    for i in range(_NBLK - _NBUF, _NBLK):
        wr[i].wait()

    tail_workers = (table_hbm.shape[0] - _MAIN) // _TAIL

    @pl.when(wid < tail_workers)
    def _():
        tb = _MAIN + wid * _TAIL
        stage = bufs[0].at[pl.ds(0, _TAIL)]
        pltpu.sync_copy(table_hbm.at[pl.ds(tb, _TAIL)], stage)
        pltpu.sync_copy(stage, out_hbm.at[pl.ds(tb, _TAIL)])


def kernel(embed_node):
    rows, cols = embed_node.shape
    flat_rows = rows * cols // _WIDTH
    x = embed_node.reshape(flat_rows, _WIDTH)
    mesh = plsc.VectorSubcoreMesh(
        core_axis_name="c", subcore_axis_name="s", num_cores=_NUM_CORES
    )
    fn = pl.kernel(
        _copy_body,
        out_type=jax.ShapeDtypeStruct((flat_rows, _WIDTH), embed_node.dtype),
        mesh=mesh,
        scratch_types=[
            [pltpu.VMEM((_BLK, _WIDTH), jnp.float32) for _ in range(_NBUF)],
            [pltpu.SemaphoreType.DMA for _ in range(_NBUF)],
            [pltpu.SemaphoreType.DMA for _ in range(_NBUF)],
        ],
    )
    return fn(x).reshape(rows, cols)
